# baseline (speedup 1.0000x reference)
import functools

import jax
import jax.numpy as jnp
from jax import lax
from jax.experimental import pallas as pl
from jax.experimental.pallas import tpu as pltpu
from jax.experimental.pallas import tpu_sc as plsc

_USER = 25000
_ITEM = 25000
_N = 50000
_E = 800000
_D = 64
_LAYERS = 3

_NC = 2
_NS = 16
_HALF = 25000
_PAD_HALF = 25088
_NFLAT = 2 * _PAD_HALF
_TRASH = 25056

_SUBW = 96
_NSUB = 8
_NCHUNK = 66
_CHUNK = _SUBW * _NSUB
_RPS = _NSUB * _NCHUNK
_IDXROWS = _NS * _RPS
_EPAD = _IDXROWS * _SUBW

_ZROWS = _PAD_HALF // _NS


def _layer_body(ego, col96, lidx96, val96, cnt96, out, colv, lidxv, valv,
                rowsv, acc, cntv,
                gidx0, gidx1, gidx2, gidx3, sidx0, sidx1, sidx2, sidx3,
                sem_g, sem_s, sem_t, sem_z):
    c = lax.axis_index("c")
    s = lax.axis_index("s")
    gidx = (gidx0, gidx1, gidx2, gidx3)
    sidx = (sidx0, sidx1, sidx2, sidx3)

    def _idx_copy(dst, src2d, q, j):
        for k in range(_SUBW // 16):
            dst[pl.ds(k * 16, 16)] = src2d[q, j, pl.ds(k * 16, 16)]

    def g_issue(q, j, b):
        _idx_copy(gidx[b], colv, q, j)
        pltpu.async_copy(
            ego.at[gidx[b]], rowsv.at[pl.ds(b * _SUBW, _SUBW)],
            sem_g.at[b],
        )

    def g_wait(b):
        pltpu.make_async_copy(
            ego.at[gidx[b]], rowsv.at[pl.ds(b * _SUBW, _SUBW)],
            sem_g.at[b],
        ).wait()

    def s_issue(b, q, j):
        _idx_copy(sidx[b], lidxv, q, j)
        pltpu.async_copy(
            rowsv.at[pl.ds(b * _SUBW, _SUBW)], acc.at[sidx[b]],
            sem_s.at[b], add=True,
        )

    def s_wait(b):
        pltpu.make_async_copy(
            rowsv.at[pl.ds(b * _SUBW, _SUBW)], acc.at[sidx[b]],
            sem_s.at[b],
        ).wait()

    def stage_issue(t, q):
        r0 = c * _IDXROWS + s * _RPS + t * _NSUB
        pltpu.async_copy(col96.at[pl.ds(r0, _NSUB)], colv.at[q], sem_t.at[q])
        pltpu.async_copy(lidx96.at[pl.ds(r0, _NSUB)], lidxv.at[q],
                         sem_t.at[q])
        pltpu.async_copy(val96.at[pl.ds(r0, _NSUB)], valv.at[q], sem_t.at[q])

    def stage_wait(q):
        for src, dst in ((col96, colv.at[q]), (lidx96, lidxv.at[q]),
                         (val96, valv.at[q])):
            pltpu.make_async_copy(src.at[pl.ds(0, _NSUB)], dst,
                                  sem_t.at[q]).wait()

    def scale(b, q, sub):
        def body(k, carry):
            vv = valv[q, sub, pl.ds(k * 16, 16)]
            for l in range(16):
                v = vv[l]
                r = b * _SUBW + k * 16 + l
                for jj in range(_D // 16):
                    rowsv[r, pl.ds(jj * 16, 16)] = (
                        rowsv[r, pl.ds(jj * 16, 16)] * v
                    )
            return carry

        lax.fori_loop(0, _SUBW // 16, body, 0)

    zero16 = jnp.zeros((16,), jnp.float32)

    def _zbody(i, carry):
        for j in range(_D // 16):
            rowsv[i, pl.ds(j * 16, 16)] = zero16
        return carry

    lax.fori_loop(0, 4 * _SUBW, _zbody, 0)

    zbase = s * _ZROWS
    nz = _ZROWS // (4 * _SUBW)
    zdescs = []
    for qq in range(nz):
        zdescs.append(pltpu.async_copy(
            rowsv.at[pl.ds(0, 4 * _SUBW)],
            acc.at[pl.ds(zbase + qq * 4 * _SUBW, 4 * _SUBW)], sem_z))
    zrem = _ZROWS - nz * 4 * _SUBW
    zdescs.append(pltpu.async_copy(
        rowsv.at[pl.ds(0, zrem)],
        acc.at[pl.ds(zbase + _ZROWS - zrem, zrem)], sem_z))

    pltpu.sync_copy(cnt96.at[c * _NS + s], cntv)
    cvec = cntv[pl.ds(0, 16)]
    npairs = cvec[0]

    stage_issue(0, 0)
    for d in zdescs:
        d.wait()
    stage_wait(0)
    plsc.subcore_barrier()

    s_issue(2, 0, _NSUB - 2)
    s_issue(3, 0, _NSUB - 1)
    g_issue(0, 0, 0)
    g_issue(0, 1, 1)

    def _pair(m, carry):
        for half in range(2):
            t = 2 * m + half
            q = half
            for sub in range(_NSUB):
                b = sub % 4
                b2 = (sub + 2) % 4
                g_wait(b)
                scale(b, q, sub)
                s_issue(b, q, sub)
                s_wait(b2)
                if sub < _NSUB - 2:
                    g_issue(q, sub + 2, b2)
                elif half == 0:
                    if sub == _NSUB - 2:
                        stage_wait(1)
                    g_issue(1, sub - (_NSUB - 2), b2)
                else:
                    @pl.when(m < npairs - 1)
                    def _():
                        if sub == _NSUB - 2:
                            stage_wait(0)
                        g_issue(0, sub - (_NSUB - 2), b2)
                if sub == 1:
                    if half == 0:
                        stage_issue(t + 1, 1)
                    else:
                        @pl.when(m < npairs - 1)
                        def _():
                            stage_issue(t + 1, 0)
        return carry

    lax.fori_loop(0, npairs, _pair, 0)

    s_wait(2)
    s_wait(3)
    plsc.subcore_barrier()

    pltpu.sync_copy(
        acc.at[pl.ds(s * _ZROWS, _ZROWS)],
        out.at[pl.ds(c * _PAD_HALF + s * _ZROWS, _ZROWS)],
    )


_layer = functools.partial(
    pl.kernel,
    out_type=jax.ShapeDtypeStruct((_NFLAT, _D), jnp.float32),
    mesh=plsc.VectorSubcoreMesh(
        core_axis_name="c", subcore_axis_name="s", num_cores=_NC,
        num_subcores=_NS,
    ),
    scratch_types=[
        pltpu.VMEM((2, _NSUB, _SUBW), jnp.int32),
        pltpu.VMEM((2, _NSUB, _SUBW), jnp.int32),
        pltpu.VMEM((2, _NSUB, _SUBW), jnp.float32),
        pltpu.VMEM((4 * _SUBW, _D), jnp.float32),
        pltpu.VMEM_SHARED((_PAD_HALF, _D), jnp.float32),
        pltpu.VMEM((16,), jnp.int32),
        pltpu.VMEM((_SUBW,), jnp.int32),
        pltpu.VMEM((_SUBW,), jnp.int32),
        pltpu.VMEM((_SUBW,), jnp.int32),
        pltpu.VMEM((_SUBW,), jnp.int32),
        pltpu.VMEM((_SUBW,), jnp.int32),
        pltpu.VMEM((_SUBW,), jnp.int32),
        pltpu.VMEM((_SUBW,), jnp.int32),
        pltpu.VMEM((_SUBW,), jnp.int32),
        pltpu.SemaphoreType.DMA((4,)),
        pltpu.SemaphoreType.DMA((4,)),
        pltpu.SemaphoreType.DMA((2,)),
        pltpu.SemaphoreType.DMA,
    ],
    compiler_params=pltpu.CompilerParams(use_tc_tiling_on_sc=False),
)(_layer_body)


def _mean_body(a, b, c, d, o):
    o[...] = (a[...] + b[...] + c[...] + d[...]) * 0.25


_mean = pl.pallas_call(
    _mean_body,
    out_shape=jax.ShapeDtypeStruct((_NFLAT, _D), jnp.float32),
    grid=(98,),
    in_specs=[pl.BlockSpec((512, _D), lambda i: (i, 0))] * 4,
    out_specs=pl.BlockSpec((512, _D), lambda i: (i, 0)),
)


def kernel(user_emb, item_emb, edge_val, edge_row, edge_col):
    cola = jnp.where(edge_col >= _HALF, edge_col + (_PAD_HALF - _HALF),
                     edge_col)

    col2 = jnp.zeros((2 * _IDXROWS * _SUBW,), jnp.int32)
    lidx2 = jnp.zeros((2 * _IDXROWS * _SUBW,), jnp.int32)
    val2 = jnp.zeros((2 * _IDXROWS * _SUBW,), jnp.float32)
    counts = []
    for core in range(2):
        m = (edge_row >= _HALF) if core else (edge_row < _HALF)
        rank = jnp.cumsum(m.astype(jnp.int32)) - 1
        sidx = rank % _NS
        within = rank // _NS
        flat = (core * _IDXROWS + sidx * _RPS + within // _SUBW) * _SUBW \
            + within % _SUBW
        dest = jnp.where(m, flat, 2 * _IDXROWS * _SUBW)
        col2 = col2.at[dest].set(cola, mode="drop")
        lidx2 = lidx2.at[dest].set(edge_row - core * _HALF, mode="drop")
        val2 = val2.at[dest].set(edge_val, mode="drop")
        n = rank[-1] + 1
        n_s = n // _NS + (jnp.arange(_NS) < n % _NS).astype(jnp.int32)
        pair_edges = 2 * _CHUNK
        counts.append(jnp.maximum(1, (n_s + pair_edges - 1) // pair_edges))
    cnt96 = jnp.broadcast_to(
        jnp.concatenate(counts)[:, None], (2 * _NS, 16)
    ).astype(jnp.int32)
    col96 = col2.reshape(2 * _IDXROWS, _SUBW)
    lidx96 = lidx2.reshape(2 * _IDXROWS, _SUBW)
    val96 = val2.reshape(2 * _IDXROWS, _SUBW)

    zpad = jnp.zeros((_PAD_HALF - _HALF, _D), jnp.float32)
    ego = jnp.concatenate([user_emb, zpad, item_emb, zpad], axis=0)

    embs = [ego]
    for _ in range(_LAYERS):
        ego = _layer(ego, col96, lidx96, val96, cnt96)
        embs.append(ego)

    mean = _mean(*embs)
    return (mean[:_USER], mean[_PAD_HALF:_PAD_HALF + _ITEM])


# speedup vs baseline: 6.4799x; 6.4799x over previous
import functools

import jax
import jax.numpy as jnp
from jax import lax
from jax.experimental import pallas as pl
from jax.experimental.pallas import tpu as pltpu
from jax.experimental.pallas import tpu_sc as plsc

_USER = 25000
_ITEM = 25000
_N = 50000
_E = 800000
_D = 64
_LAYERS = 3

_NC = 2
_NS = 16
_HALF = 25000
_PAD_HALF = 25088
_NFLAT = 2 * _PAD_HALF
_TRASH = 25056

_SUBW = 128
_NSUB = 6
_NBUF = 3
_NCHUNK = 66
_CHUNK = _SUBW * _NSUB
_RPS = _NSUB * _NCHUNK
_IDXROWS = _NS * _RPS
_EPAD = _IDXROWS * _SUBW

_ZROWS = _PAD_HALF // _NS


def _layer_body(ego, col96, lidx96, val96, out, colv, lidxv, valv, rowsv, acc,
                gidx0, gidx1, gidx2, sidx0, sidx1, sidx2,
                sem_g, sem_s, sem_t, sem_z):
    c = lax.axis_index("c")
    s = lax.axis_index("s")
    gidx = (gidx0, gidx1, gidx2)
    sidx = (sidx0, sidx1, sidx2)

    def _idx_copy(dst, src2d, q, j):
        for k in range(_SUBW // 16):
            dst[pl.ds(k * 16, 16)] = src2d[q, j, pl.ds(k * 16, 16)]

    def g_issue(q, j, b):
        _idx_copy(gidx[b], colv, q, j)
        pltpu.async_copy(
            ego.at[gidx[b]], rowsv.at[pl.ds(b * _SUBW, _SUBW)],
            sem_g.at[b],
        )

    def g_wait(b):
        pltpu.make_async_copy(
            ego.at[gidx[b]], rowsv.at[pl.ds(b * _SUBW, _SUBW)],
            sem_g.at[b],
        ).wait()

    def s_issue(b, q, j):
        _idx_copy(sidx[b], lidxv, q, j)
        pltpu.async_copy(
            rowsv.at[pl.ds(b * _SUBW, _SUBW)], acc.at[sidx[b]],
            sem_s.at[b], add=True,
        )

    def s_wait(b):
        pltpu.make_async_copy(
            rowsv.at[pl.ds(b * _SUBW, _SUBW)], acc.at[sidx[b]],
            sem_s.at[b],
        ).wait()

    def stage_issue(t, q):
        r0 = s * _RPS + t * _NSUB
        pltpu.async_copy(col96.at[pl.ds(r0, _NSUB)], colv.at[q], sem_t.at[q])
        pltpu.async_copy(
            lidx96.at[pl.ds(c * _IDXROWS + r0, _NSUB)], lidxv.at[q],
            sem_t.at[q],
        )
        pltpu.async_copy(val96.at[pl.ds(r0, _NSUB)], valv.at[q], sem_t.at[q])

    def stage_wait(q):
        for src, dst in ((col96, colv.at[q]), (lidx96, lidxv.at[q]),
                         (val96, valv.at[q])):
            pltpu.make_async_copy(src.at[pl.ds(0, _NSUB)], dst,
                                  sem_t.at[q]).wait()

    def scale(b, q, sub):
        def body(k, carry):
            vv = valv[q, sub, pl.ds(k * 16, 16)]
            for l in range(16):
                v = vv[l]
                r = b * _SUBW + k * 16 + l
                for jj in range(_D // 16):
                    rowsv[r, pl.ds(jj * 16, 16)] = (
                        rowsv[r, pl.ds(jj * 16, 16)] * v
                    )
            return carry

        lax.fori_loop(0, _SUBW // 16, body, 0)

    zero16 = jnp.zeros((16,), jnp.float32)

    def _zbody(i, carry):
        for j in range(_D // 16):
            rowsv[i, pl.ds(j * 16, 16)] = zero16
        return carry

    lax.fori_loop(0, _NBUF * _SUBW, _zbody, 0)

    zbase = s * _ZROWS
    nz = _ZROWS // (_NBUF * _SUBW)
    zdescs = []
    for qq in range(nz):
        zdescs.append(pltpu.async_copy(
            rowsv.at[pl.ds(0, _NBUF * _SUBW)],
            acc.at[pl.ds(zbase + qq * _NBUF * _SUBW, _NBUF * _SUBW)], sem_z))
    zrem = _ZROWS - nz * _NBUF * _SUBW
    zdescs.append(pltpu.async_copy(
        rowsv.at[pl.ds(0, zrem)],
        acc.at[pl.ds(zbase + _ZROWS - zrem, zrem)], sem_z))

    stage_issue(0, 0)
    for d in zdescs:
        d.wait()
    stage_wait(0)
    plsc.subcore_barrier()

    s_issue(2, 0, _NSUB - 1)
    g_issue(0, 0, 0)
    g_issue(0, 1, 1)

    def _pair(m, carry):
        for half in range(2):
            t = 2 * m + half
            q = half
            for sub in range(_NSUB):
                b = sub % _NBUF
                b2 = (sub + 2) % _NBUF
                g_wait(b)
                scale(b, q, sub)
                s_issue(b, q, sub)
                s_wait(b2)
                if sub < _NSUB - 2:
                    g_issue(q, sub + 2, b2)
                elif half == 0:
                    if sub == _NSUB - 2:
                        stage_wait(1)
                    g_issue(1, sub - (_NSUB - 2), b2)
                else:
                    @pl.when(m < (_NCHUNK // 2) - 1)
                    def _():
                        if sub == _NSUB - 2:
                            stage_wait(0)
                        g_issue(0, sub - (_NSUB - 2), b2)
                if sub == 1:
                    if half == 0:
                        stage_issue(t + 1, 1)
                    else:
                        @pl.when(m < (_NCHUNK // 2) - 1)
                        def _():
                            stage_issue(t + 1, 0)
        return carry

    lax.fori_loop(0, _NCHUNK // 2, _pair, 0)

    s_wait(2)
    plsc.subcore_barrier()

    pltpu.sync_copy(
        acc.at[pl.ds(s * _ZROWS, _ZROWS)],
        out.at[pl.ds(c * _PAD_HALF + s * _ZROWS, _ZROWS)],
    )


_layer = functools.partial(
    pl.kernel,
    out_type=jax.ShapeDtypeStruct((_NFLAT, _D), jnp.float32),
    mesh=plsc.VectorSubcoreMesh(
        core_axis_name="c", subcore_axis_name="s", num_cores=_NC,
        num_subcores=_NS,
    ),
    scratch_types=[
        pltpu.VMEM((2, _NSUB, _SUBW), jnp.int32),
        pltpu.VMEM((2, _NSUB, _SUBW), jnp.int32),
        pltpu.VMEM((2, _NSUB, _SUBW), jnp.float32),
        pltpu.VMEM((_NBUF * _SUBW, _D), jnp.float32),
        pltpu.VMEM_SHARED((_PAD_HALF, _D), jnp.float32),
        pltpu.VMEM((_SUBW,), jnp.int32),
        pltpu.VMEM((_SUBW,), jnp.int32),
        pltpu.VMEM((_SUBW,), jnp.int32),
        pltpu.VMEM((_SUBW,), jnp.int32),
        pltpu.VMEM((_SUBW,), jnp.int32),
        pltpu.VMEM((_SUBW,), jnp.int32),
        pltpu.SemaphoreType.DMA((_NBUF,)),
        pltpu.SemaphoreType.DMA((_NBUF,)),
        pltpu.SemaphoreType.DMA((2,)),
        pltpu.SemaphoreType.DMA,
    ],
    compiler_params=pltpu.CompilerParams(use_tc_tiling_on_sc=False),
)(_layer_body)


def _mean_body(a, b, c, d, o):
    o[...] = (a[...] + b[...] + c[...] + d[...]) * 0.25


_mean = pl.pallas_call(
    _mean_body,
    out_shape=jax.ShapeDtypeStruct((_NFLAT, _D), jnp.float32),
    grid=(98,),
    in_specs=[pl.BlockSpec((512, _D), lambda i: (i, 0))] * 4,
    out_specs=pl.BlockSpec((512, _D), lambda i: (i, 0)),
)


def kernel(user_emb, item_emb, edge_val, edge_row, edge_col):
    pad = _EPAD - _E
    colp = jnp.concatenate([edge_col, jnp.zeros((pad,), jnp.int32)])
    rowp = jnp.concatenate([edge_row, jnp.zeros((pad,), jnp.int32)])
    valp = jnp.concatenate([edge_val, jnp.zeros((pad,), jnp.float32)])

    col96 = jnp.where(colp >= _HALF, colp + (_PAD_HALF - _HALF), colp)
    col96 = col96.reshape(_IDXROWS, _SUBW)
    lidx0 = jnp.where(rowp < _HALF, rowp, _TRASH)
    lidx1 = jnp.where(rowp >= _HALF, rowp - _HALF, _TRASH)
    lidx96 = jnp.concatenate([lidx0, lidx1]).reshape(2 * _IDXROWS, _SUBW)
    val96 = valp.reshape(_IDXROWS, _SUBW)

    zpad = jnp.zeros((_PAD_HALF - _HALF, _D), jnp.float32)
    ego = jnp.concatenate([user_emb, zpad, item_emb, zpad], axis=0)

    embs = [ego]
    for _ in range(_LAYERS):
        ego = _layer(ego, col96, lidx96, val96)
        embs.append(ego)

    mean = _mean(*embs)
    return (mean[:_USER], mean[_PAD_HALF:_PAD_HALF + _ITEM])
